# baseline (speedup 1.0000x reference)
import functools

import jax
import jax.numpy as jnp
from jax import lax
from jax.experimental import pallas as pl
from jax.experimental.pallas import tpu as pltpu
from jax.experimental.pallas import tpu_sc as plsc

_C = 32
_K = 8192
_H = _K // 2
_TN = 256


def _argmin_body(x_ref, e_ref, idx_ref, loss_ref, et_ref, esq_ref):
    i = pl.program_id(0)

    @pl.when(i == 0)
    def _init():
        e = e_ref[...]
        esq_ref[...] = jnp.sum(e * e, axis=0, keepdims=True)
        loss_ref[0, 0] = 0.0
        et_ref[...] = jnp.transpose(e, (1, 0))

    xb = x_ref[...]
    conv = lax.dot_general((2.0 * xb).astype(jnp.bfloat16),
                           e_ref[...].astype(jnp.bfloat16),
                           (((1,), (0,)), ((), ())),
                           preferred_element_type=jnp.float32)
    xsq = jnp.sum(xb * xb, axis=1, keepdims=True)
    d = (xsq - conv) + esq_ref[...]
    d1 = d[:, :_H]
    d2 = d[:, _H:]
    m1 = jnp.min(d1, axis=1, keepdims=True)
    m2 = jnp.min(d2, axis=1, keepdims=True)
    iota = lax.broadcasted_iota(jnp.int32, (d.shape[0], _H), 1)
    a1 = jnp.min(jnp.where(d1 == m1, iota, _H), axis=1)
    a2 = jnp.min(jnp.where(d2 == m2, iota, _H), axis=1) + _H
    m1v = m1[:, 0]
    m2v = m2[:, 0]
    win2 = m2v < m1v.astype(jnp.bfloat16).astype(jnp.float32)
    idx_ref[...] = jnp.where(win2, a2, a1)
    loss_ref[0, 0] += jnp.sum(jnp.where(win2, m2v, m1v))

    @pl.when(i == pl.num_programs(0) - 1)
    def _final():
        loss_ref[0, 0] = loss_ref[0, 0] * (1.0 / (_TN * pl.num_programs(0) * _C))


def _tc_argmin(flat_x, embeddings):
    n = flat_x.shape[0]
    grid = n // _TN
    return pl.pallas_call(
        _argmin_body,
        grid=(grid,),
        in_specs=[
            pl.BlockSpec((_TN, _C), lambda i: (i, 0)),
            pl.BlockSpec((_C, _K), lambda i: (0, 0)),
        ],
        out_specs=[
            pl.BlockSpec((_TN,), lambda i: (i,)),
            pl.BlockSpec(memory_space=pltpu.SMEM),
            pl.BlockSpec((_K, _C), lambda i: (0, 0)),
        ],
        out_shape=[
            jax.ShapeDtypeStruct((n,), jnp.int32),
            jax.ShapeDtypeStruct((1, 1), jnp.float32),
            jax.ShapeDtypeStruct((_K, _C), jnp.float32),
        ],
        scratch_shapes=[pltpu.VMEM((1, _K), jnp.float32)],
        compiler_params=pltpu.CompilerParams(
            dimension_semantics=("arbitrary",)),
    )(flat_x, embeddings)


@functools.cache
def _make_sc_gather(v, d, b):
    info = plsc.get_sparse_core_info()
    nc, ns = info.num_cores, info.num_subcores
    nw = nc * ns
    b_per_w = b // nw
    mesh = plsc.VectorSubcoreMesh(core_axis_name="c", subcore_axis_name="s")

    @functools.partial(
        pl.kernel, mesh=mesh,
        out_type=jax.ShapeDtypeStruct((b, d), jnp.float32),
        scratch_types=[
            pltpu.VMEM((b_per_w,), jnp.int32),
            pltpu.VMEM((b_per_w, d), jnp.float32),
            pltpu.SemaphoreType.DMA,
        ],
        compiler_params=pltpu.CompilerParams(use_tc_tiling_on_sc=False),
    )
    def gather_k(table_hbm, idx_hbm, out_hbm, idx_v, rows_v, sem):
        wid = lax.axis_index("s") * nc + lax.axis_index("c")
        base = wid * b_per_w
        pltpu.sync_copy(idx_hbm.at[pl.ds(base, b_per_w)], idx_v)
        pltpu.async_copy(table_hbm.at[idx_v], rows_v, sem).wait()
        pltpu.sync_copy(rows_v, out_hbm.at[pl.ds(base, b_per_w)])

    return gather_k


def kernel(x, embeddings):
    b, c, h, w = x.shape
    n = b * h * w
    flat_x = jnp.transpose(x, (0, 2, 3, 1)).reshape(n, c)
    idx, loss_sum, table = _tc_argmin(flat_x, embeddings)
    loss = loss_sum[0, 0]
    q_flat = _make_sc_gather(_K, c, n)(table, idx)
    quantized = jnp.transpose(q_flat.reshape(b, h, w, c), (0, 3, 1, 2))
    return quantized, loss, idx.reshape(b, -1)


# speedup vs baseline: 1.0572x; 1.0572x over previous
import functools

import jax
import jax.numpy as jnp
from jax import lax
from jax.experimental import pallas as pl
from jax.experimental.pallas import tpu as pltpu
from jax.experimental.pallas import tpu_sc as plsc

_C = 32
_K = 8192
_H = _K // 2
_TN = 256


def _argmin_body(x_ref, e_ref, idx_ref, loss_ref, et_ref, esq_ref):
    i = pl.program_id(0)

    @pl.when(i == 0)
    def _init():
        e = e_ref[...]
        esq_ref[...] = jnp.sum(e * e, axis=0, keepdims=True)
        loss_ref[0, 0] = 0.0
        et_ref[...] = jnp.transpose(e, (1, 0))

    xb = x_ref[...]
    conv = lax.dot_general((2.0 * xb).astype(jnp.bfloat16),
                           e_ref[...].astype(jnp.bfloat16),
                           (((1,), (0,)), ((), ())),
                           preferred_element_type=jnp.float32)
    xsq = jnp.sum(xb * xb, axis=1, keepdims=True)
    d = (xsq - conv) + esq_ref[...]
    d1 = d[:, :_H]
    d2 = d[:, _H:]
    m1v = jnp.min(d1, axis=1)
    m2v = jnp.min(d2, axis=1)
    a1 = jnp.argmin(d1, axis=1)
    a2 = jnp.argmin(d2, axis=1) + _H
    win2 = m2v < m1v.astype(jnp.bfloat16).astype(jnp.float32)
    idx_ref[...] = jnp.where(win2, a2, a1)
    loss_ref[0, 0] += jnp.sum(jnp.where(win2, m2v, m1v))

    @pl.when(i == pl.num_programs(0) - 1)
    def _final():
        loss_ref[0, 0] = loss_ref[0, 0] * (1.0 / (_TN * pl.num_programs(0) * _C))


def _tc_argmin(flat_x, embeddings):
    n = flat_x.shape[0]
    grid = n // _TN
    return pl.pallas_call(
        _argmin_body,
        grid=(grid,),
        in_specs=[
            pl.BlockSpec((_TN, _C), lambda i: (i, 0)),
            pl.BlockSpec((_C, _K), lambda i: (0, 0)),
        ],
        out_specs=[
            pl.BlockSpec((_TN,), lambda i: (i,)),
            pl.BlockSpec(memory_space=pltpu.SMEM),
            pl.BlockSpec((_K, _C), lambda i: (0, 0)),
        ],
        out_shape=[
            jax.ShapeDtypeStruct((n,), jnp.int32),
            jax.ShapeDtypeStruct((1, 1), jnp.float32),
            jax.ShapeDtypeStruct((_K, _C), jnp.float32),
        ],
        scratch_shapes=[pltpu.VMEM((1, _K), jnp.float32)],
        compiler_params=pltpu.CompilerParams(
            dimension_semantics=("arbitrary",)),
    )(flat_x, embeddings)


@functools.cache
def _make_sc_gather(v, d, b):
    info = plsc.get_sparse_core_info()
    nc, ns = info.num_cores, info.num_subcores
    nw = nc * ns
    b_per_w = b // nw
    mesh = plsc.VectorSubcoreMesh(core_axis_name="c", subcore_axis_name="s")

    @functools.partial(
        pl.kernel, mesh=mesh,
        out_type=jax.ShapeDtypeStruct((b, d), jnp.float32),
        scratch_types=[
            pltpu.VMEM((b_per_w,), jnp.int32),
            pltpu.VMEM((b_per_w, d), jnp.float32),
            pltpu.SemaphoreType.DMA,
        ],
        compiler_params=pltpu.CompilerParams(use_tc_tiling_on_sc=False),
    )
    def gather_k(table_hbm, idx_hbm, out_hbm, idx_v, rows_v, sem):
        wid = lax.axis_index("s") * nc + lax.axis_index("c")
        base = wid * b_per_w
        pltpu.sync_copy(idx_hbm.at[pl.ds(base, b_per_w)], idx_v)
        pltpu.async_copy(table_hbm.at[idx_v], rows_v, sem).wait()
        pltpu.sync_copy(rows_v, out_hbm.at[pl.ds(base, b_per_w)])

    return gather_k


def kernel(x, embeddings):
    b, c, h, w = x.shape
    n = b * h * w
    flat_x = jnp.transpose(x, (0, 2, 3, 1)).reshape(n, c)
    idx, loss_sum, table = _tc_argmin(flat_x, embeddings)
    loss = loss_sum[0, 0]
    q_flat = _make_sc_gather(_K, c, n)(table, idx)
    quantized = jnp.transpose(q_flat.reshape(b, h, w, c), (0, 3, 1, 2))
    return quantized, loss, idx.reshape(b, -1)


# speedup vs baseline: 1.1792x; 1.1155x over previous
import functools

import jax
import jax.numpy as jnp
from jax import lax
from jax.experimental import pallas as pl
from jax.experimental.pallas import tpu as pltpu
from jax.experimental.pallas import tpu_sc as plsc

_C = 32
_K = 8192
_H = _K // 2
_TN = 256


def _argmin_body(x_ref, e_ref, idx_ref, loss_ref, et_ref, esq_ref):
    i = pl.program_id(0)

    @pl.when(i == 0)
    def _init():
        e = e_ref[...]
        esq_ref[...] = jnp.sum(e * e, axis=0, keepdims=True)
        loss_ref[0, 0] = 0.0
        et_ref[...] = jnp.transpose(e, (1, 0))

    xb = x_ref[...]
    conv = lax.dot_general((2.0 * xb).astype(jnp.bfloat16),
                           e_ref[...].astype(jnp.bfloat16),
                           (((1,), (0,)), ((), ())),
                           preferred_element_type=jnp.float32)
    xsq = jnp.sum(xb * xb, axis=1, keepdims=True)
    esq = esq_ref[...]
    lane = lax.broadcasted_iota(jnp.int32, (_TN, 128), 1)

    def half_argmin(off):
        runv = (xsq - conv[:, off:off + 128]) + esq[:, off:off + 128]
        runi = jnp.zeros((_TN, 128), jnp.int32)
        for ci in range(1, _H // 128):
            o = off + ci * 128
            dc = (xsq - conv[:, o:o + 128]) + esq[:, o:o + 128]
            upd = dc < runv
            runi = jnp.where(upd, ci, runi)
            runv = jnp.where(upd, dc, runv)
        m = jnp.min(runv, axis=1, keepdims=True)
        kk = runi * 128 + lane
        a = jnp.min(jnp.where(runv == m, kk, _K), axis=1)
        return m[:, 0], a

    m1v, a1 = half_argmin(0)
    m2v, a2 = half_argmin(_H)
    a2 = a2 + _H
    win2 = m2v < m1v.astype(jnp.bfloat16).astype(jnp.float32)
    idx_ref[...] = jnp.where(win2, a2, a1)
    loss_ref[0, 0] += jnp.sum(jnp.where(win2, m2v, m1v))

    @pl.when(i == pl.num_programs(0) - 1)
    def _final():
        loss_ref[0, 0] = loss_ref[0, 0] * (1.0 / (_TN * pl.num_programs(0) * _C))


def _tc_argmin(flat_x, embeddings):
    n = flat_x.shape[0]
    grid = n // _TN
    return pl.pallas_call(
        _argmin_body,
        grid=(grid,),
        in_specs=[
            pl.BlockSpec((_TN, _C), lambda i: (i, 0)),
            pl.BlockSpec((_C, _K), lambda i: (0, 0)),
        ],
        out_specs=[
            pl.BlockSpec((_TN,), lambda i: (i,)),
            pl.BlockSpec(memory_space=pltpu.SMEM),
            pl.BlockSpec((_K, _C), lambda i: (0, 0)),
        ],
        out_shape=[
            jax.ShapeDtypeStruct((n,), jnp.int32),
            jax.ShapeDtypeStruct((1, 1), jnp.float32),
            jax.ShapeDtypeStruct((_K, _C), jnp.float32),
        ],
        scratch_shapes=[pltpu.VMEM((1, _K), jnp.float32)],
        compiler_params=pltpu.CompilerParams(
            dimension_semantics=("arbitrary",)),
    )(flat_x, embeddings)


@functools.cache
def _make_sc_gather(v, d, b):
    info = plsc.get_sparse_core_info()
    nc, ns = info.num_cores, info.num_subcores
    nw = nc * ns
    b_per_w = b // nw
    mesh = plsc.VectorSubcoreMesh(core_axis_name="c", subcore_axis_name="s")

    @functools.partial(
        pl.kernel, mesh=mesh,
        out_type=jax.ShapeDtypeStruct((b, d), jnp.float32),
        scratch_types=[
            pltpu.VMEM((b_per_w,), jnp.int32),
            pltpu.VMEM((b_per_w, d), jnp.float32),
            pltpu.SemaphoreType.DMA,
        ],
        compiler_params=pltpu.CompilerParams(use_tc_tiling_on_sc=False),
    )
    def gather_k(table_hbm, idx_hbm, out_hbm, idx_v, rows_v, sem):
        wid = lax.axis_index("s") * nc + lax.axis_index("c")
        base = wid * b_per_w
        pltpu.sync_copy(idx_hbm.at[pl.ds(base, b_per_w)], idx_v)
        pltpu.async_copy(table_hbm.at[idx_v], rows_v, sem).wait()
        pltpu.sync_copy(rows_v, out_hbm.at[pl.ds(base, b_per_w)])

    return gather_k


def kernel(x, embeddings):
    b, c, h, w = x.shape
    n = b * h * w
    flat_x = jnp.transpose(x, (0, 2, 3, 1)).reshape(n, c)
    idx, loss_sum, table = _tc_argmin(flat_x, embeddings)
    loss = loss_sum[0, 0]
    q_flat = _make_sc_gather(_K, c, n)(table, idx)
    quantized = jnp.transpose(q_flat.reshape(b, h, w, c), (0, 3, 1, 2))
    return quantized, loss, idx.reshape(b, -1)
